# baseline (device time: 186654 ns/iter reference)
import jax
import jax.numpy as jnp
from jax import lax
from jax.experimental import pallas as pl
from jax.experimental.pallas import tpu as pltpu

N_DEV = 4
SQ = 1024
SKV = 1024
HQ = 8
DH = 128
D = HQ * DH
SCALE = 0.08838834764831843
W = D + DH


def kernel(x, Wq, K_ext, V_ext, Wo):
    x2 = x.reshape(SQ, D)
    K2 = K_ext.reshape(SKV, D)
    V2 = V_ext.reshape(SKV, D)

    def body(x_ref, wq_ref, k_ref, v_ref, wo_ref, out_ref,
             q_ref, acc_ref, comm_ref, send_sems, recv_sems):
        my = lax.axis_index("i")
        left = (my - 1) % N_DEV
        right = (my + 1) % N_DEV

        barrier = pltpu.get_barrier_semaphore()
        for nbr in (left, right):
            pl.semaphore_signal(barrier, inc=1, device_id=(nbr,),
                                device_id_type=pl.DeviceIdType.MESH)
        pl.semaphore_wait(barrier, 2)

        q_ref[...] = jnp.dot(x_ref[...], wq_ref[...],
                             preferred_element_type=jnp.float32)

        qi = lax.broadcasted_iota(jnp.int32, (SQ, SKV), 0)
        ki = lax.broadcasted_iota(jnp.int32, (SQ, SKV), 1)
        mask = ((qi // 64) % 4) == ((ki // 64) % 4)

        local = comm_ref.at[0]
        for h in range(HQ):
            c0, c1 = h * DH, (h + 1) * DH
            s = lax.dot_general(q_ref[:, c0:c1], k_ref[:, c0:c1],
                                (((1,), (1,)), ((), ())),
                                preferred_element_type=jnp.float32)
            w = jnp.where(mask, jnp.exp(s * SCALE), 0.0)
            local[:, c0:c1] = jnp.dot(w, v_ref[:, c0:c1],
                                      preferred_element_type=jnp.float32)
            local[:, D + h:D + h + 1] = jnp.sum(w, axis=1, keepdims=True)

        acc_ref[...] = local[...]

        for hop in range(N_DEV - 1):
            rdma = pltpu.make_async_remote_copy(
                src_ref=comm_ref.at[hop],
                dst_ref=comm_ref.at[hop + 1],
                send_sem=send_sems.at[hop],
                recv_sem=recv_sems.at[hop],
                device_id=(right,),
                device_id_type=pl.DeviceIdType.MESH,
            )
            rdma.start()
            rdma.wait()
            acc_ref[...] += comm_ref[hop + 1]

        for h in range(HQ):
            c0, c1 = h * DH, (h + 1) * DH
            q_ref[:, c0:c1] = acc_ref[:, c0:c1] / acc_ref[:, D + h:D + h + 1]
        out_ref[...] = jnp.dot(q_ref[...], wo_ref[...],
                               preferred_element_type=jnp.float32)

    out = pl.pallas_call(
        body,
        out_shape=jax.ShapeDtypeStruct((SQ, D), jnp.float32),
        in_specs=[pl.BlockSpec(memory_space=pltpu.VMEM)] * 5,
        out_specs=pl.BlockSpec(memory_space=pltpu.VMEM),
        scratch_shapes=[
            pltpu.VMEM((SQ, D), jnp.float32),
            pltpu.VMEM((SQ, W), jnp.float32),
            pltpu.VMEM((N_DEV, SQ, W), jnp.float32),
            pltpu.SemaphoreType.DMA((N_DEV - 1,)),
            pltpu.SemaphoreType.DMA((N_DEV - 1,)),
        ],
        compiler_params=pltpu.CompilerParams(collective_id=0),
    )(x2, Wq, K2, V2, Wo)
    return out.reshape(1, SQ, D)


# device time: 65558 ns/iter; 2.8472x vs baseline; 2.8472x over previous
import jax
import jax.numpy as jnp
from jax import lax
from jax.experimental import pallas as pl
from jax.experimental.pallas import tpu as pltpu

N_DEV = 4
SQ = 1024
SKV = 1024
HQ = 8
DH = 128
D = HQ * DH
CH = 256
HALF = CH // 2
SCALE = 0.08838834764831843
W = D + DH


def kernel(x, Wq, K_ext, V_ext, Wo):
    x2 = x.reshape(SQ, D)
    K2 = K_ext.reshape(SKV, D)
    V2 = V_ext.reshape(SKV, D)

    def body(x_ref, wq_ref, k_ref, v_ref, wo_ref, out_ref,
             xp_ref, qp_ref, kr_ref, vr_ref, part_ref,
             rsr_ref, rsl_ref, og_ref, rs_sems, ag_sems):
        my = lax.axis_index("i")
        left = (my - 1) % N_DEV
        right = (my + 1) % N_DEV

        barrier = pltpu.get_barrier_semaphore()
        for nbr in (left, right):
            pl.semaphore_signal(barrier, inc=1, device_id=(nbr,),
                                device_id_type=pl.DeviceIdType.MESH)
        pl.semaphore_wait(barrier, 2)

        for r in range(4):
            for u in range(4):
                xp_ref[256 * r + 64 * u:256 * r + 64 * u + 64, :] = (
                    x_ref[64 * (4 * u + r):64 * (4 * u + r) + 64, :])
        qp_ref[...] = jnp.dot(xp_ref[...], wq_ref[...],
                              preferred_element_type=jnp.float32)

        for k in range(4):
            c = (my + k) % N_DEV
            for u in range(4):
                kv0 = 256 * u + 64 * c
                kr_ref[64 * u:64 * u + 64, :] = k_ref[pl.ds(kv0, 64), :]
                vr_ref[64 * u:64 * u + 64, :] = v_ref[pl.ds(kv0, 64), :]
            qc = qp_ref[pl.ds(CH * c, CH), :]
            for h in range(HQ):
                c0, c1 = h * DH, (h + 1) * DH
                s = lax.dot_general(qc[:, c0:c1], kr_ref[:, c0:c1],
                                    (((1,), (1,)), ((), ())),
                                    preferred_element_type=jnp.float32)
                w = jnp.exp(s * SCALE)
                part_ref[k, :, c0:c1] = jnp.dot(
                    w, vr_ref[:, c0:c1], preferred_element_type=jnp.float32)
                part_ref[k, :, D + h:D + h + 1] = jnp.sum(
                    w, axis=1, keepdims=True)

        UP = pl.ds(0, HALF)
        LO = pl.ds(HALF, HALF)

        ADD_R = (2, 1, 0)
        ADD_L = (2, 3, 0)
        for t in range(3):
            src_r = part_ref.at[3, UP] if t == 0 else rsr_ref.at[t - 1]
            src_l = part_ref.at[1, LO] if t == 0 else rsl_ref.at[t - 1]
            rd_r = pltpu.make_async_remote_copy(
                src_ref=src_r, dst_ref=rsr_ref.at[t],
                send_sem=rs_sems.at[0, t], recv_sem=rs_sems.at[1, t],
                device_id=(right,), device_id_type=pl.DeviceIdType.MESH)
            rd_l = pltpu.make_async_remote_copy(
                src_ref=src_l, dst_ref=rsl_ref.at[t],
                send_sem=rs_sems.at[2, t], recv_sem=rs_sems.at[3, t],
                device_id=(left,), device_id_type=pl.DeviceIdType.MESH)
            rd_r.start()
            rd_l.start()
            rd_r.wait()
            rd_l.wait()
            rsr_ref[t, :, :] = rsr_ref[t] + part_ref[ADD_R[t], 0:HALF, :]
            rsl_ref[t, :, :] = rsl_ref[t] + part_ref[ADD_L[t], HALF:CH, :]

        for h in range(HQ):
            c0, c1 = h * DH, (h + 1) * DH
            kr_ref[0:HALF, c0:c1] = (
                rsr_ref[2, :, c0:c1] / rsr_ref[2, :, D + h:D + h + 1])
            kr_ref[HALF:CH, c0:c1] = (
                rsl_ref[2, :, c0:c1] / rsl_ref[2, :, D + h:D + h + 1])
        og_ref[0, :, :] = jnp.dot(kr_ref[...], wo_ref[...],
                                  preferred_element_type=jnp.float32)

        SEND_R = (0, 3, 2)
        RECV_R = (3, 2, 1)
        SEND_L = (0, 1, 2)
        RECV_L = (1, 2, 3)
        for t in range(3):
            ag_r = pltpu.make_async_remote_copy(
                src_ref=og_ref.at[SEND_R[t], UP],
                dst_ref=og_ref.at[RECV_R[t], UP],
                send_sem=ag_sems.at[0, t], recv_sem=ag_sems.at[1, t],
                device_id=(right,), device_id_type=pl.DeviceIdType.MESH)
            ag_l = pltpu.make_async_remote_copy(
                src_ref=og_ref.at[SEND_L[t], LO],
                dst_ref=og_ref.at[RECV_L[t], LO],
                send_sem=ag_sems.at[2, t], recv_sem=ag_sems.at[3, t],
                device_id=(left,), device_id_type=pl.DeviceIdType.MESH)
            ag_r.start()
            ag_l.start()
            ag_r.wait()
            ag_l.wait()

        for k in range(4):
            c = (my + k) % N_DEV
            for u in range(4):
                out_ref[pl.ds(256 * u + 64 * c, 64), :] = (
                    og_ref[k, 64 * u:64 * u + 64, :])

    out = pl.pallas_call(
        body,
        out_shape=jax.ShapeDtypeStruct((SQ, D), jnp.float32),
        in_specs=[pl.BlockSpec(memory_space=pltpu.VMEM)] * 5,
        out_specs=pl.BlockSpec(memory_space=pltpu.VMEM),
        scratch_shapes=[
            pltpu.VMEM((SQ, D), jnp.float32),
            pltpu.VMEM((SQ, D), jnp.float32),
            pltpu.VMEM((CH, D), jnp.float32),
            pltpu.VMEM((CH, D), jnp.float32),
            pltpu.VMEM((4, CH, W), jnp.float32),
            pltpu.VMEM((3, HALF, W), jnp.float32),
            pltpu.VMEM((3, HALF, W), jnp.float32),
            pltpu.VMEM((4, CH, D), jnp.float32),
            pltpu.SemaphoreType.DMA((4, 3)),
            pltpu.SemaphoreType.DMA((4, 3)),
        ],
        compiler_params=pltpu.CompilerParams(collective_id=0),
    )(x2, Wq, K2, V2, Wo)
    return out.reshape(1, SQ, D)


# device time: 45088 ns/iter; 4.1398x vs baseline; 1.4540x over previous
import jax
import jax.numpy as jnp
from jax import lax
from jax.experimental import pallas as pl
from jax.experimental.pallas import tpu as pltpu

N_DEV = 4
SQ = 1024
SKV = 1024
HQ = 8
DH = 128
D = HQ * DH
CH = 256
HALF = CH // 2
SCALE = 0.08838834764831843
W = D + DH
BF16 = jnp.bfloat16


def kernel(x, Wq, K_ext, V_ext, Wo):
    x2 = x.reshape(SQ, D)
    K2 = K_ext.reshape(SKV, D)
    V2 = V_ext.reshape(SKV, D)

    def body(x_ref, wq_ref, k_ref, v_ref, wo_ref, out_ref,
             xp_ref, kr_ref, vr_ref, part_ref,
             rsr_ref, rsl_ref, og_ref, rs_sems, ag_sems):
        my = lax.axis_index("i")
        left = (my - 1) % N_DEV
        right = (my + 1) % N_DEV

        barrier = pltpu.get_barrier_semaphore()
        for nbr in (left, right):
            pl.semaphore_signal(barrier, inc=1, device_id=(nbr,),
                                device_id_type=pl.DeviceIdType.MESH)
        pl.semaphore_wait(barrier, 2)

        for r in range(4):
            for u in range(4):
                xp_ref[256 * r + 64 * u:256 * r + 64 * u + 64, :] = (
                    x_ref[64 * (4 * u + r):64 * (4 * u + r) + 64, :])

        def compute_chunk(k):
            c = (my + k) % N_DEV
            qc = jnp.dot(xp_ref[pl.ds(CH * c, CH), :], wq_ref[...],
                         preferred_element_type=jnp.float32)
            for u in range(4):
                kv0 = 256 * u + 64 * c
                kr_ref[64 * u:64 * u + 64, :] = k_ref[pl.ds(kv0, 64), :]
                vr_ref[64 * u:64 * u + 64, :] = v_ref[pl.ds(kv0, 64), :]
            for h in range(HQ):
                c0, c1 = h * DH, (h + 1) * DH
                s = lax.dot_general(qc[:, c0:c1], kr_ref[:, c0:c1],
                                    (((1,), (1,)), ((), ())),
                                    preferred_element_type=jnp.float32)
                w = jnp.exp(s * SCALE)
                part_ref[k, :, c0:c1] = jnp.dot(
                    w, vr_ref[:, c0:c1],
                    preferred_element_type=jnp.float32).astype(BF16)
                part_ref[k, :, D + h:D + h + 1] = jnp.sum(
                    w, axis=1, keepdims=True).astype(BF16)

        UP = pl.ds(0, HALF)
        LO = pl.ds(HALF, HALF)

        RS_ADD_R = (2, 1, 0)
        RS_ADD_L = (2, 3, 0)

        def start_rs(t, src_r, src_l):
            rd_r = pltpu.make_async_remote_copy(
                src_ref=src_r, dst_ref=rsr_ref.at[t],
                send_sem=rs_sems.at[0, t], recv_sem=rs_sems.at[1, t],
                device_id=(right,), device_id_type=pl.DeviceIdType.MESH)
            rd_l = pltpu.make_async_remote_copy(
                src_ref=src_l, dst_ref=rsl_ref.at[t],
                send_sem=rs_sems.at[2, t], recv_sem=rs_sems.at[3, t],
                device_id=(left,), device_id_type=pl.DeviceIdType.MESH)
            rd_r.start()
            rd_l.start()
            return rd_r, rd_l

        def rs_add(t):
            rsr_ref[t, :, :] = rsr_ref[t] + part_ref[RS_ADD_R[t], 0:HALF, :]
            rsl_ref[t, :, :] = rsl_ref[t] + part_ref[RS_ADD_L[t], HALF:CH, :]

        pend = []
        compute_chunk(3)
        compute_chunk(1)
        r0 = start_rs(0, part_ref.at[3, UP], part_ref.at[1, LO])
        pend += r0
        compute_chunk(2)
        r0[0].wait_recv()
        r0[1].wait_recv()
        rs_add(0)
        r1 = start_rs(1, rsr_ref.at[0], rsl_ref.at[0])
        pend += r1
        compute_chunk(0)
        r1[0].wait_recv()
        r1[1].wait_recv()
        rs_add(1)
        r2 = start_rs(2, rsr_ref.at[1], rsl_ref.at[1])
        pend += r2
        r2[0].wait_recv()
        r2[1].wait_recv()
        rs_add(2)

        for h in range(HQ):
            c0, c1 = h * DH, (h + 1) * DH
            kr_ref[0:HALF, c0:c1] = (
                rsr_ref[2, :, c0:c1].astype(jnp.float32)
                / rsr_ref[2, :, D + h:D + h + 1].astype(jnp.float32))
            kr_ref[HALF:CH, c0:c1] = (
                rsl_ref[2, :, c0:c1].astype(jnp.float32)
                / rsl_ref[2, :, D + h:D + h + 1].astype(jnp.float32))
        og_ref[0, :, :] = jnp.dot(
            kr_ref[...], wo_ref[...],
            preferred_element_type=jnp.float32).astype(BF16)

        SEND_R = (0, 3, 2)
        RECV_R = (3, 2, 1)
        SEND_L = (0, 1, 2)
        RECV_L = (1, 2, 3)

        def store_slot(k_slot, us):
            c = (my + k_slot) % N_DEV
            for u in us:
                out_ref[pl.ds(256 * u + 64 * c, 64), :] = (
                    og_ref[k_slot, 64 * u:64 * u + 64, :].astype(jnp.float32))

        for t in range(3):
            ag_r = pltpu.make_async_remote_copy(
                src_ref=og_ref.at[SEND_R[t], UP],
                dst_ref=og_ref.at[RECV_R[t], UP],
                send_sem=ag_sems.at[0, t], recv_sem=ag_sems.at[1, t],
                device_id=(right,), device_id_type=pl.DeviceIdType.MESH)
            ag_l = pltpu.make_async_remote_copy(
                src_ref=og_ref.at[SEND_L[t], LO],
                dst_ref=og_ref.at[RECV_L[t], LO],
                send_sem=ag_sems.at[2, t], recv_sem=ag_sems.at[3, t],
                device_id=(left,), device_id_type=pl.DeviceIdType.MESH)
            ag_r.start()
            ag_l.start()
            pend += [ag_r, ag_l]
            if t == 0:
                store_slot(0, (0, 1, 2, 3))
            else:
                store_slot(RECV_R[t - 1], (0, 1))
                store_slot(RECV_L[t - 1], (2, 3))
            ag_r.wait_recv()
            ag_l.wait_recv()
        store_slot(RECV_R[2], (0, 1))
        store_slot(RECV_L[2], (2, 3))

        for d in pend:
            d.wait_send()

    out = pl.pallas_call(
        body,
        out_shape=jax.ShapeDtypeStruct((SQ, D), jnp.float32),
        in_specs=[pl.BlockSpec(memory_space=pltpu.VMEM)] * 5,
        out_specs=pl.BlockSpec(memory_space=pltpu.VMEM),
        scratch_shapes=[
            pltpu.VMEM((SQ, D), jnp.float32),
            pltpu.VMEM((CH, D), jnp.float32),
            pltpu.VMEM((CH, D), jnp.float32),
            pltpu.VMEM((4, CH, W), BF16),
            pltpu.VMEM((3, HALF, W), BF16),
            pltpu.VMEM((3, HALF, W), BF16),
            pltpu.VMEM((4, CH, D), BF16),
            pltpu.SemaphoreType.DMA((4, 3)),
            pltpu.SemaphoreType.DMA((4, 3)),
        ],
        compiler_params=pltpu.CompilerParams(collective_id=0),
    )(x2, Wq, K2, V2, Wo)
    return out.reshape(1, SQ, D)


# device time: 42734 ns/iter; 4.3678x vs baseline; 1.0551x over previous
import jax
import jax.numpy as jnp
from jax import lax
from jax.experimental import pallas as pl
from jax.experimental.pallas import tpu as pltpu

N_DEV = 4
SQ = 1024
SKV = 1024
HQ = 8
DH = 128
D = HQ * DH
CH = 256
HALF = CH // 2
SCALE = 0.08838834764831843
W = D + DH
BF16 = jnp.bfloat16


def kernel(x, Wq, K_ext, V_ext, Wo):
    x2 = x.reshape(SQ, D)
    K2 = K_ext.reshape(SKV, D)
    V2 = V_ext.reshape(SKV, D)

    def body(x_ref, wq_ref, k_ref, v_ref, wo_ref, out_ref,
             xp_ref, kr_ref, vr_ref, part_ref,
             rsr_ref, rsl_ref, og_ref, rs_sems, ag_sems):
        my = lax.axis_index("i")
        left = (my - 1) % N_DEV
        right = (my + 1) % N_DEV

        barrier = pltpu.get_barrier_semaphore()
        for nbr in (left, right):
            pl.semaphore_signal(barrier, inc=1, device_id=(nbr,),
                                device_id_type=pl.DeviceIdType.MESH)
        pl.semaphore_wait(barrier, 2)

        for r in range(4):
            for u in range(4):
                xp_ref[256 * r + 64 * u:256 * r + 64 * u + 64, :] = (
                    x_ref[64 * (4 * u + r):64 * (4 * u + r) + 64, :])

        def gather_kv(k):
            c = (my + k) % N_DEV
            for u in range(4):
                kv0 = 256 * u + 64 * c
                kr_ref[k, 64 * u:64 * u + 64, :] = k_ref[pl.ds(kv0, 64), :]
                vr_ref[k, 64 * u:64 * u + 64, :] = v_ref[pl.ds(kv0, 64), :]

        def compute_half(k, up):
            c = (my + k) % N_DEV
            off = 0 if up else HALF
            rows = slice(off, off + HALF)
            q = jnp.dot(xp_ref[pl.ds(CH * c + off, HALF), :], wq_ref[...],
                        preferred_element_type=jnp.float32)
            for h in range(HQ):
                c0, c1 = h * DH, (h + 1) * DH
                s = lax.dot_general(q[:, c0:c1], kr_ref[k, :, c0:c1],
                                    (((1,), (1,)), ((), ())),
                                    preferred_element_type=jnp.float32)
                w = jnp.exp(s * SCALE)
                part_ref[k, rows, c0:c1] = jnp.dot(
                    w, vr_ref[k, :, c0:c1],
                    preferred_element_type=jnp.float32).astype(BF16)
                part_ref[k, rows, D + h:D + h + 1] = jnp.sum(
                    w, axis=1, keepdims=True).astype(BF16)

        UP = pl.ds(0, HALF)
        LO = pl.ds(HALF, HALF)

        RS_ADD_R = (2, 1, 0)
        RS_ADD_L = (2, 3, 0)

        def start_rs(dirn, t, src):
            to = right if dirn == 0 else left
            dst = rsr_ref if dirn == 0 else rsl_ref
            rd = pltpu.make_async_remote_copy(
                src_ref=src, dst_ref=dst.at[t],
                send_sem=rs_sems.at[2 * dirn, t],
                recv_sem=rs_sems.at[2 * dirn + 1, t],
                device_id=(to,), device_id_type=pl.DeviceIdType.MESH)
            rd.start()
            return rd

        pend = []

        gather_kv(3)
        compute_half(3, True)
        r0r = start_rs(0, 0, part_ref.at[3, UP])
        gather_kv(1)
        compute_half(1, False)
        r0l = start_rs(1, 0, part_ref.at[1, LO])
        pend += [r0r, r0l]

        gather_kv(2)
        compute_half(2, True)
        compute_half(2, False)

        r0r.wait_recv()
        rsr_ref[0, :, :] = rsr_ref[0] + part_ref[RS_ADD_R[0], 0:HALF, :]
        r1r = start_rs(0, 1, rsr_ref.at[0])
        r0l.wait_recv()
        rsl_ref[0, :, :] = rsl_ref[0] + part_ref[RS_ADD_L[0], HALF:CH, :]
        r1l = start_rs(1, 1, rsl_ref.at[0])
        pend += [r1r, r1l]

        compute_half(1, True)
        compute_half(3, False)

        r1r.wait_recv()
        rsr_ref[1, :, :] = rsr_ref[1] + part_ref[RS_ADD_R[1], 0:HALF, :]
        r2r = start_rs(0, 2, rsr_ref.at[1])
        r1l.wait_recv()
        rsl_ref[1, :, :] = rsl_ref[1] + part_ref[RS_ADD_L[1], HALF:CH, :]
        r2l = start_rs(1, 2, rsl_ref.at[1])
        pend += [r2r, r2l]

        gather_kv(0)
        compute_half(0, True)
        compute_half(0, False)

        r2r.wait_recv()
        rsr_ref[2, :, :] = rsr_ref[2] + part_ref[RS_ADD_R[2], 0:HALF, :]
        r2l.wait_recv()
        rsl_ref[2, :, :] = rsl_ref[2] + part_ref[RS_ADD_L[2], HALF:CH, :]

        def ag_send(idx, src, dst, to):
            rd = pltpu.make_async_remote_copy(
                src_ref=src, dst_ref=dst,
                send_sem=ag_sems.at[0, idx], recv_sem=ag_sems.at[1, idx],
                device_id=(to,), device_id_type=pl.DeviceIdType.MESH)
            rd.start()
            return rd

        def norm_half(row_sel, src):
            for h in range(HQ):
                c0, c1 = h * DH, (h + 1) * DH
                kr_ref[0, row_sel, c0:c1] = (
                    src[:, c0:c1].astype(jnp.float32)
                    / src[:, D + h:D + h + 1].astype(jnp.float32))

        norm_half(slice(0, HALF), rsr_ref[2])
        og_ref[0, 0:HALF, :] = jnp.dot(
            kr_ref[0, 0:HALF, :], wo_ref[...],
            preferred_element_type=jnp.float32).astype(BF16)
        a1ru = ag_send(0, og_ref.at[0, UP], og_ref.at[3, UP], right)
        a1lu = ag_send(1, og_ref.at[0, UP], og_ref.at[1, UP], left)

        norm_half(slice(HALF, CH), rsl_ref[2])
        og_ref[0, HALF:CH, :] = jnp.dot(
            kr_ref[0, HALF:CH, :], wo_ref[...],
            preferred_element_type=jnp.float32).astype(BF16)
        a1rl = ag_send(2, og_ref.at[0, LO], og_ref.at[3, LO], right)
        a1ll = ag_send(3, og_ref.at[0, LO], og_ref.at[1, LO], left)
        pend += [a1ru, a1lu, a1rl, a1ll]

        def store_slot(k_slot, us):
            c = (my + k_slot) % N_DEV
            for u in us:
                out_ref[pl.ds(256 * u + 64 * c, 64), :] = (
                    og_ref[k_slot, 64 * u:64 * u + 64, :].astype(jnp.float32))

        store_slot(0, (0, 1, 2, 3))

        a1ru.wait_recv()
        a1lu.wait_recv()
        a2l = ag_send(4, og_ref.at[1, UP], og_ref.at[2, UP], left)
        a1rl.wait_recv()
        a1ll.wait_recv()
        a2r = ag_send(5, og_ref.at[3, LO], og_ref.at[2, LO], right)
        pend += [a2l, a2r]

        store_slot(1, (0, 1, 2, 3))
        store_slot(3, (0, 1, 2, 3))

        a2l.wait_recv()
        a2r.wait_recv()
        store_slot(2, (0, 1, 2, 3))

        for d in pend:
            d.wait_send()

    out = pl.pallas_call(
        body,
        out_shape=jax.ShapeDtypeStruct((SQ, D), jnp.float32),
        in_specs=[pl.BlockSpec(memory_space=pltpu.VMEM)] * 5,
        out_specs=pl.BlockSpec(memory_space=pltpu.VMEM),
        scratch_shapes=[
            pltpu.VMEM((SQ, D), jnp.float32),
            pltpu.VMEM((4, CH, D), jnp.float32),
            pltpu.VMEM((4, CH, D), jnp.float32),
            pltpu.VMEM((4, CH, W), BF16),
            pltpu.VMEM((3, HALF, W), BF16),
            pltpu.VMEM((3, HALF, W), BF16),
            pltpu.VMEM((4, CH, D), BF16),
            pltpu.SemaphoreType.DMA((4, 3)),
            pltpu.SemaphoreType.DMA((2, 6)),
        ],
        compiler_params=pltpu.CompilerParams(collective_id=0),
    )(x2, Wq, K2, V2, Wo)
    return out.reshape(1, SQ, D)
